# Initial kernel scaffold; baseline (speedup 1.0000x reference)
import jax
import jax.numpy as jnp
from jax.experimental import pallas as pl


def kernel(fields, sides, species, moves, items, abilities, move_attributes, pokemon_attributes, species_table, move_table, item_table, ability_table):
    raise NotImplementedError("write your pallas kernel here")


# speedup vs baseline: 2.5516x; 2.5516x over previous
import functools

import jax
import jax.numpy as jnp
from jax import lax
from jax.experimental import pallas as pl
from jax.experimental.pallas import tpu as pltpu
from jax.experimental.pallas import tpu_sc as plsc

B = 16384
S = B * 2 * 6
NC, NS = 2, 16
NW = NC * NS
SPW = S // NW
C = 128
NCHUNK = SPW // C

D_SP, D_MV, D_IT, D_AB = 128, 64, 32, 32
D_OUT = 512

def _kernel_kwargs():
    mesh = plsc.VectorSubcoreMesh(core_axis_name="c", subcore_axis_name="s",
                                  num_cores=NC, num_subcores=NS)
    return dict(
        out_type=jax.ShapeDtypeStruct((S, D_OUT), jnp.float32),
        mesh=mesh,
        scratch_types=[
            pltpu.VMEM((C,), jnp.int32),
            pltpu.VMEM((C,), jnp.int32),
            pltpu.VMEM((C,), jnp.int32),
            pltpu.VMEM((C,), jnp.int32),
            pltpu.VMEM((C,), jnp.int32),
            pltpu.VMEM((C,), jnp.int32),
            pltpu.VMEM((C,), jnp.int32),
            pltpu.VMEM((C, D_SP), jnp.float32),
            pltpu.VMEM((C, D_IT), jnp.float32),
            pltpu.VMEM((C, D_AB), jnp.float32),
            pltpu.VMEM((C, D_MV), jnp.float32),
            pltpu.VMEM((C, D_MV), jnp.float32),
            pltpu.VMEM((C, D_MV), jnp.float32),
            pltpu.VMEM((C, D_MV), jnp.float32),
            pltpu.VMEM((C, 32), jnp.float32),
            pltpu.VMEM((C, 32), jnp.float32),
            pltpu.SemaphoreType.DMA,
        ],
        compiler_params=pltpu.CompilerParams(use_tc_tiling_on_sc=False),
    )


def _encode_body(sp_idx, it_idx, ab_idx, m0_idx, m1_idx, m2_idx, m3_idx,
                 ma, pa, sp_tab, mv_tab, it_tab, ab_tab, out,
                 i_sp, i_it, i_ab, i_m0, i_m1, i_m2, i_m3,
                 r_sp, r_it, r_ab, r_m0, r_m1, r_m2, r_m3, b_ma, b_pa, sem):
    wid = lax.axis_index("s") * NC + lax.axis_index("c")
    w0 = wid * SPW

    def chunk(i, carry):
        base = w0 + i * C
        row = pl.ds(base, C)
        pltpu.sync_copy(sp_idx.at[row], i_sp)
        pltpu.sync_copy(it_idx.at[row], i_it)
        pltpu.sync_copy(ab_idx.at[row], i_ab)
        pltpu.sync_copy(m0_idx.at[row], i_m0)
        pltpu.sync_copy(m1_idx.at[row], i_m1)
        pltpu.sync_copy(m2_idx.at[row], i_m2)
        pltpu.sync_copy(m3_idx.at[row], i_m3)
        pltpu.sync_copy(ma.at[row], b_ma)
        pltpu.sync_copy(pa.at[row], b_pa)
        pltpu.async_copy(sp_tab.at[i_sp], r_sp, sem).wait()
        pltpu.async_copy(it_tab.at[i_it], r_it, sem).wait()
        pltpu.async_copy(ab_tab.at[i_ab], r_ab, sem).wait()
        pltpu.async_copy(mv_tab.at[i_m0], r_m0, sem).wait()
        pltpu.async_copy(mv_tab.at[i_m1], r_m1, sem).wait()
        pltpu.async_copy(mv_tab.at[i_m2], r_m2, sem).wait()
        pltpu.async_copy(mv_tab.at[i_m3], r_m3, sem).wait()
        pltpu.sync_copy(r_sp, out.at[row, pl.ds(0, D_SP)])
        pltpu.sync_copy(r_it, out.at[row, pl.ds(128, D_IT)])
        pltpu.sync_copy(r_ab, out.at[row, pl.ds(160, D_AB)])
        pltpu.sync_copy(r_m0, out.at[row, pl.ds(192, D_MV)])
        pltpu.sync_copy(r_m1, out.at[row, pl.ds(256, D_MV)])
        pltpu.sync_copy(r_m2, out.at[row, pl.ds(320, D_MV)])
        pltpu.sync_copy(r_m3, out.at[row, pl.ds(384, D_MV)])
        pltpu.sync_copy(b_ma, out.at[row, pl.ds(448, 32)])
        pltpu.sync_copy(b_pa, out.at[row, pl.ds(480, 32)])
        return carry

    lax.fori_loop(0, NCHUNK, chunk, 0)


@functools.cache
def _encode():
    return pl.kernel(_encode_body, **_kernel_kwargs())


def kernel(fields, sides, species, moves, items, abilities,
           move_attributes, pokemon_attributes,
           species_table, move_table, item_table, ability_table):
    sp = species.reshape(S).astype(jnp.int32)
    it = items.reshape(S).astype(jnp.int32)
    ab = abilities.reshape(S).astype(jnp.int32)
    mv = moves.reshape(S, 4).astype(jnp.int32)
    ma = move_attributes.reshape(S, 32)
    pa = pokemon_attributes.reshape(S, 32)
    concat = _encode()(sp, it, ab, mv[:, 0], mv[:, 1], mv[:, 2], mv[:, 3],
                       ma, pa, species_table, move_table, item_table,
                       ability_table)
    return (fields, sides, concat.reshape(B, 2, 6, D_OUT))


# speedup vs baseline: 2.9723x; 1.1649x over previous
import functools

import jax
import jax.numpy as jnp
from jax import lax
from jax.experimental import pallas as pl
from jax.experimental.pallas import tpu as pltpu
from jax.experimental.pallas import tpu_sc as plsc

B = 16384
S = B * 2 * 6
NC, NS = 2, 16
NW = NC * NS
SPW = S // NW
C = 128
NCHUNK = SPW // C

D_SP, D_MV, D_IT, D_AB = 128, 64, 32, 32
D_OUT = 512

def _kernel_kwargs():
    mesh = plsc.VectorSubcoreMesh(core_axis_name="c", subcore_axis_name="s",
                                  num_cores=NC, num_subcores=NS)
    return dict(
        out_type=jax.ShapeDtypeStruct((S, D_OUT), jnp.float32),
        mesh=mesh,
        scratch_types=[
            pltpu.VMEM((C,), jnp.int32),
            pltpu.VMEM((C,), jnp.int32),
            pltpu.VMEM((C,), jnp.int32),
            pltpu.VMEM((C,), jnp.int32),
            pltpu.VMEM((C,), jnp.int32),
            pltpu.VMEM((C,), jnp.int32),
            pltpu.VMEM((C,), jnp.int32),
            pltpu.VMEM((C, D_SP), jnp.float32),
            pltpu.VMEM((C, D_IT), jnp.float32),
            pltpu.VMEM((C, D_AB), jnp.float32),
            pltpu.VMEM((C, D_MV), jnp.float32),
            pltpu.VMEM((C, D_MV), jnp.float32),
            pltpu.VMEM((C, D_MV), jnp.float32),
            pltpu.VMEM((C, D_MV), jnp.float32),
            pltpu.VMEM((C, 32), jnp.float32),
            pltpu.VMEM((C, 32), jnp.float32),
            pltpu.SemaphoreType.DMA,
            pltpu.SemaphoreType.DMA,
        ],
        compiler_params=pltpu.CompilerParams(use_tc_tiling_on_sc=False),
    )


def _encode_body(sp_idx, it_idx, ab_idx, m0_idx, m1_idx, m2_idx, m3_idx,
                 ma, pa, sp_tab, mv_tab, it_tab, ab_tab, out,
                 i_sp, i_it, i_ab, i_m0, i_m1, i_m2, i_m3,
                 r_sp, r_it, r_ab, r_m0, r_m1, r_m2, r_m3, b_ma, b_pa,
                 semA, semB):
    wid = lax.axis_index("s") * NC + lax.axis_index("c")
    w0 = wid * SPW

    def batch(cps):
        for cp in cps:
            cp.start()
        for cp in cps:
            cp.wait()

    def chunk(i, carry):
        base = w0 + i * C
        row = pl.ds(base, C)
        batch([
            pltpu.make_async_copy(sp_idx.at[row], i_sp, semA),
            pltpu.make_async_copy(it_idx.at[row], i_it, semA),
            pltpu.make_async_copy(ab_idx.at[row], i_ab, semA),
            pltpu.make_async_copy(m0_idx.at[row], i_m0, semA),
            pltpu.make_async_copy(m1_idx.at[row], i_m1, semA),
            pltpu.make_async_copy(m2_idx.at[row], i_m2, semA),
            pltpu.make_async_copy(m3_idx.at[row], i_m3, semA),
            pltpu.make_async_copy(ma.at[row], b_ma, semA),
            pltpu.make_async_copy(pa.at[row], b_pa, semA),
        ])
        batch([
            pltpu.make_async_copy(sp_tab.at[i_sp], r_sp, semA),
            pltpu.make_async_copy(it_tab.at[i_it], r_it, semA),
            pltpu.make_async_copy(ab_tab.at[i_ab], r_ab, semA),
            pltpu.make_async_copy(mv_tab.at[i_m0], r_m0, semA),
            pltpu.make_async_copy(mv_tab.at[i_m1], r_m1, semA),
            pltpu.make_async_copy(mv_tab.at[i_m2], r_m2, semA),
            pltpu.make_async_copy(mv_tab.at[i_m3], r_m3, semA),
        ])
        batch([
            pltpu.make_async_copy(r_sp, out.at[row, pl.ds(0, D_SP)], semB),
            pltpu.make_async_copy(r_it, out.at[row, pl.ds(128, D_IT)], semB),
            pltpu.make_async_copy(r_ab, out.at[row, pl.ds(160, D_AB)], semB),
            pltpu.make_async_copy(r_m0, out.at[row, pl.ds(192, D_MV)], semB),
            pltpu.make_async_copy(r_m1, out.at[row, pl.ds(256, D_MV)], semB),
            pltpu.make_async_copy(r_m2, out.at[row, pl.ds(320, D_MV)], semB),
            pltpu.make_async_copy(r_m3, out.at[row, pl.ds(384, D_MV)], semB),
            pltpu.make_async_copy(b_ma, out.at[row, pl.ds(448, 32)], semB),
            pltpu.make_async_copy(b_pa, out.at[row, pl.ds(480, 32)], semB),
        ])
        return carry

    lax.fori_loop(0, NCHUNK, chunk, 0)


@functools.cache
def _encode():
    return pl.kernel(_encode_body, **_kernel_kwargs())


def kernel(fields, sides, species, moves, items, abilities,
           move_attributes, pokemon_attributes,
           species_table, move_table, item_table, ability_table):
    sp = species.reshape(S).astype(jnp.int32)
    it = items.reshape(S).astype(jnp.int32)
    ab = abilities.reshape(S).astype(jnp.int32)
    mv = moves.reshape(S, 4).astype(jnp.int32)
    ma = move_attributes.reshape(S, 32)
    pa = pokemon_attributes.reshape(S, 32)
    concat = _encode()(sp, it, ab, mv[:, 0], mv[:, 1], mv[:, 2], mv[:, 3],
                       ma, pa, species_table, move_table, item_table,
                       ability_table)
    return (fields, sides, concat.reshape(B, 2, 6, D_OUT))


# speedup vs baseline: 2.9746x; 1.0008x over previous
import functools

import jax
import jax.numpy as jnp
from jax import lax
from jax.experimental import pallas as pl
from jax.experimental.pallas import tpu as pltpu
from jax.experimental.pallas import tpu_sc as plsc

B = 16384
S = B * 2 * 6
NC, NS = 2, 16
NW = NC * NS
SPW = S // NW
C = 96
NCHUNK = SPW // C
NPAIR = NCHUNK // 2

D_SP, D_MV, D_IT, D_AB = 128, 64, 32, 32
D_OUT = 512


def _buf_set():
    return [
        pltpu.VMEM((C,), jnp.int32),
        pltpu.VMEM((C,), jnp.int32),
        pltpu.VMEM((C,), jnp.int32),
        pltpu.VMEM((C,), jnp.int32),
        pltpu.VMEM((C,), jnp.int32),
        pltpu.VMEM((C,), jnp.int32),
        pltpu.VMEM((C,), jnp.int32),
        pltpu.VMEM((C, D_SP), jnp.float32),
        pltpu.VMEM((C, D_IT), jnp.float32),
        pltpu.VMEM((C, D_AB), jnp.float32),
        pltpu.VMEM((C, D_MV), jnp.float32),
        pltpu.VMEM((C, D_MV), jnp.float32),
        pltpu.VMEM((C, D_MV), jnp.float32),
        pltpu.VMEM((C, D_MV), jnp.float32),
        pltpu.VMEM((C, 32), jnp.float32),
        pltpu.VMEM((C, 32), jnp.float32),
    ]


def _kernel_kwargs():
    mesh = plsc.VectorSubcoreMesh(core_axis_name="c", subcore_axis_name="s",
                                  num_cores=NC, num_subcores=NS)
    return dict(
        out_type=jax.ShapeDtypeStruct((S, D_OUT), jnp.float32),
        mesh=mesh,
        scratch_types=[
            *_buf_set(),
            *_buf_set(),
            pltpu.SemaphoreType.DMA,
            pltpu.SemaphoreType.DMA,
            pltpu.SemaphoreType.DMA,
            pltpu.SemaphoreType.DMA,
            pltpu.SemaphoreType.DMA,
            pltpu.SemaphoreType.DMA,
        ],
        compiler_params=pltpu.CompilerParams(use_tc_tiling_on_sc=False),
    )


def _encode_body(sp_idx, it_idx, ab_idx, m0_idx, m1_idx, m2_idx, m3_idx,
                 ma, pa, sp_tab, mv_tab, it_tab, ab_tab, out, *rest):
    bufs = (rest[0:16], rest[16:32])
    semL = (rest[32], rest[33])
    semG = (rest[34], rest[35])
    semO = (rest[36], rest[37])

    wid = lax.axis_index("s") * NC + lax.axis_index("c")
    w0 = wid * SPW

    def ld_copies(c, bset, sem):
        row = pl.ds(w0 + c * C, C)
        i_sp, i_it, i_ab, i_m0, i_m1, i_m2, i_m3 = bset[0:7]
        b_ma, b_pa = bset[14:16]
        return [
            pltpu.make_async_copy(sp_idx.at[row], i_sp, sem),
            pltpu.make_async_copy(it_idx.at[row], i_it, sem),
            pltpu.make_async_copy(ab_idx.at[row], i_ab, sem),
            pltpu.make_async_copy(m0_idx.at[row], i_m0, sem),
            pltpu.make_async_copy(m1_idx.at[row], i_m1, sem),
            pltpu.make_async_copy(m2_idx.at[row], i_m2, sem),
            pltpu.make_async_copy(m3_idx.at[row], i_m3, sem),
            pltpu.make_async_copy(ma.at[row], b_ma, sem),
            pltpu.make_async_copy(pa.at[row], b_pa, sem),
        ]

    def g_copies(bset, sem):
        i_sp, i_it, i_ab, i_m0, i_m1, i_m2, i_m3 = bset[0:7]
        r_sp, r_it, r_ab, r_m0, r_m1, r_m2, r_m3 = bset[7:14]
        return [
            pltpu.make_async_copy(sp_tab.at[i_sp], r_sp, sem),
            pltpu.make_async_copy(it_tab.at[i_it], r_it, sem),
            pltpu.make_async_copy(ab_tab.at[i_ab], r_ab, sem),
            pltpu.make_async_copy(mv_tab.at[i_m0], r_m0, sem),
            pltpu.make_async_copy(mv_tab.at[i_m1], r_m1, sem),
            pltpu.make_async_copy(mv_tab.at[i_m2], r_m2, sem),
            pltpu.make_async_copy(mv_tab.at[i_m3], r_m3, sem),
        ]

    def out_copies(c, bset, sem):
        row = pl.ds(w0 + c * C, C)
        r_sp, r_it, r_ab, r_m0, r_m1, r_m2, r_m3 = bset[7:14]
        b_ma, b_pa = bset[14:16]
        return [
            pltpu.make_async_copy(r_sp, out.at[row, pl.ds(0, D_SP)], sem),
            pltpu.make_async_copy(r_it, out.at[row, pl.ds(128, D_IT)], sem),
            pltpu.make_async_copy(r_ab, out.at[row, pl.ds(160, D_AB)], sem),
            pltpu.make_async_copy(r_m0, out.at[row, pl.ds(192, D_MV)], sem),
            pltpu.make_async_copy(r_m1, out.at[row, pl.ds(256, D_MV)], sem),
            pltpu.make_async_copy(r_m2, out.at[row, pl.ds(320, D_MV)], sem),
            pltpu.make_async_copy(r_m3, out.at[row, pl.ds(384, D_MV)], sem),
            pltpu.make_async_copy(b_ma, out.at[row, pl.ds(448, 32)], sem),
            pltpu.make_async_copy(b_pa, out.at[row, pl.ds(480, 32)], sem),
        ]

    def fire(cps):
        for cp in cps:
            cp.start()

    def drain(cps):
        for cp in cps:
            cp.wait()

    fire(ld_copies(0, bufs[0], semL[0]))

    def pair(k, carry):
        c0 = 2 * k
        c1 = c0 + 1

        drain(ld_copies(c0, bufs[0], semL[0]))
        fire(g_copies(bufs[0], semG[0]))

        @pl.when(k > 0)
        def _():
            drain(out_copies(c0 - 1, bufs[1], semO[1]))
        fire(ld_copies(c1, bufs[1], semL[1]))
        drain(g_copies(bufs[0], semG[0]))
        fire(out_copies(c0, bufs[0], semO[0]))

        drain(ld_copies(c1, bufs[1], semL[1]))
        fire(g_copies(bufs[1], semG[1]))
        drain(out_copies(c0, bufs[0], semO[0]))

        @pl.when(k < NPAIR - 1)
        def _():
            fire(ld_copies(c0 + 2, bufs[0], semL[0]))
        drain(g_copies(bufs[1], semG[1]))
        fire(out_copies(c1, bufs[1], semO[1]))
        return carry

    lax.fori_loop(0, NPAIR, pair, 0)
    drain(out_copies(NCHUNK - 1, bufs[1], semO[1]))


@functools.cache
def _encode():
    return pl.kernel(_encode_body, **_kernel_kwargs())


def kernel(fields, sides, species, moves, items, abilities,
           move_attributes, pokemon_attributes,
           species_table, move_table, item_table, ability_table):
    sp = species.reshape(S).astype(jnp.int32)
    it = items.reshape(S).astype(jnp.int32)
    ab = abilities.reshape(S).astype(jnp.int32)
    mv = moves.reshape(S, 4).astype(jnp.int32)
    ma = move_attributes.reshape(S, 32)
    pa = pokemon_attributes.reshape(S, 32)
    concat = _encode()(sp, it, ab, mv[:, 0], mv[:, 1], mv[:, 2], mv[:, 3],
                       ma, pa, species_table, move_table, item_table,
                       ability_table)
    return (fields, sides, concat.reshape(B, 2, 6, D_OUT))


# speedup vs baseline: 3.1862x; 1.0711x over previous
import functools

import jax
import jax.numpy as jnp
from jax import lax
from jax.experimental import pallas as pl
from jax.experimental.pallas import tpu as pltpu
from jax.experimental.pallas import tpu_sc as plsc

B = 16384
S = B * 2 * 6
NC, NS = 2, 16
NW = NC * NS
SPW = S // NW
C = 128
NCHUNK = SPW // C

D_SP, D_MV, D_IT, D_AB = 128, 64, 32, 32
D_OUT = 512

V_SP, V_MV, V_IT, V_AB = 1152, 1024, 1024, 384
R_SP, R_MV, R_IT, R_AB = V_SP // NS, V_MV // NS, V_IT // NS, V_AB // NS


def _kernel_kwargs():
    mesh = plsc.VectorSubcoreMesh(core_axis_name="c", subcore_axis_name="s",
                                  num_cores=NC, num_subcores=NS)
    return dict(
        out_type=jax.ShapeDtypeStruct((S, D_OUT), jnp.float32),
        mesh=mesh,
        scratch_types=[
            pltpu.VMEM((C,), jnp.int32),
            pltpu.VMEM((C,), jnp.int32),
            pltpu.VMEM((C,), jnp.int32),
            pltpu.VMEM((C,), jnp.int32),
            pltpu.VMEM((C,), jnp.int32),
            pltpu.VMEM((C,), jnp.int32),
            pltpu.VMEM((C,), jnp.int32),
            pltpu.VMEM((C, D_SP), jnp.float32),
            pltpu.VMEM((C, D_IT), jnp.float32),
            pltpu.VMEM((C, D_AB), jnp.float32),
            pltpu.VMEM((C, D_MV), jnp.float32),
            pltpu.VMEM((C, D_MV), jnp.float32),
            pltpu.VMEM((C, D_MV), jnp.float32),
            pltpu.VMEM((C, D_MV), jnp.float32),
            pltpu.VMEM((C, 32), jnp.float32),
            pltpu.VMEM((C, 32), jnp.float32),
            pltpu.VMEM_SHARED((V_SP, D_SP), jnp.float32),
            pltpu.VMEM_SHARED((V_MV, D_MV), jnp.float32),
            pltpu.VMEM_SHARED((V_IT, D_IT), jnp.float32),
            pltpu.VMEM_SHARED((V_AB, D_AB), jnp.float32),
            pltpu.SemaphoreType.DMA,
            pltpu.SemaphoreType.DMA,
        ],
        compiler_params=pltpu.CompilerParams(use_tc_tiling_on_sc=False),
    )


def _encode_body(sp_idx, it_idx, ab_idx, m0_idx, m1_idx, m2_idx, m3_idx,
                 ma, pa, sp_tab, mv_tab, it_tab, ab_tab, out,
                 i_sp, i_it, i_ab, i_m0, i_m1, i_m2, i_m3,
                 r_sp, r_it, r_ab, r_m0, r_m1, r_m2, r_m3, b_ma, b_pa,
                 s_sp, s_mv, s_it, s_ab, semA, semB):
    sid = lax.axis_index("s")
    wid = sid * NC + lax.axis_index("c")
    w0 = wid * SPW

    def batch(cps):
        for cp in cps:
            cp.start()
        for cp in cps:
            cp.wait()

    for tab_hbm, tab_s, rpt, buf in (
            (sp_tab, s_sp, R_SP, r_sp), (mv_tab, s_mv, R_MV, r_m0),
            (it_tab, s_it, R_IT, r_it), (ab_tab, s_ab, R_AB, r_ab)):
        rows = pl.ds(sid * rpt, rpt)
        pltpu.sync_copy(tab_hbm.at[rows], buf.at[pl.ds(0, rpt)])
        pltpu.sync_copy(buf.at[pl.ds(0, rpt)], tab_s.at[rows])
    plsc.subcore_barrier()

    def chunk(i, carry):
        base = w0 + i * C
        row = pl.ds(base, C)
        batch([
            pltpu.make_async_copy(sp_idx.at[row], i_sp, semA),
            pltpu.make_async_copy(it_idx.at[row], i_it, semA),
            pltpu.make_async_copy(ab_idx.at[row], i_ab, semA),
            pltpu.make_async_copy(m0_idx.at[row], i_m0, semA),
            pltpu.make_async_copy(m1_idx.at[row], i_m1, semA),
            pltpu.make_async_copy(m2_idx.at[row], i_m2, semA),
            pltpu.make_async_copy(m3_idx.at[row], i_m3, semA),
            pltpu.make_async_copy(ma.at[row], b_ma, semA),
            pltpu.make_async_copy(pa.at[row], b_pa, semA),
        ])
        batch([
            pltpu.make_async_copy(s_sp.at[i_sp], r_sp, semA),
            pltpu.make_async_copy(s_it.at[i_it], r_it, semA),
            pltpu.make_async_copy(s_ab.at[i_ab], r_ab, semA),
            pltpu.make_async_copy(s_mv.at[i_m0], r_m0, semA),
            pltpu.make_async_copy(s_mv.at[i_m1], r_m1, semA),
            pltpu.make_async_copy(s_mv.at[i_m2], r_m2, semA),
            pltpu.make_async_copy(s_mv.at[i_m3], r_m3, semA),
        ])
        batch([
            pltpu.make_async_copy(r_sp, out.at[row, pl.ds(0, D_SP)], semB),
            pltpu.make_async_copy(r_it, out.at[row, pl.ds(128, D_IT)], semB),
            pltpu.make_async_copy(r_ab, out.at[row, pl.ds(160, D_AB)], semB),
            pltpu.make_async_copy(r_m0, out.at[row, pl.ds(192, D_MV)], semB),
            pltpu.make_async_copy(r_m1, out.at[row, pl.ds(256, D_MV)], semB),
            pltpu.make_async_copy(r_m2, out.at[row, pl.ds(320, D_MV)], semB),
            pltpu.make_async_copy(r_m3, out.at[row, pl.ds(384, D_MV)], semB),
            pltpu.make_async_copy(b_ma, out.at[row, pl.ds(448, 32)], semB),
            pltpu.make_async_copy(b_pa, out.at[row, pl.ds(480, 32)], semB),
        ])
        return carry

    lax.fori_loop(0, NCHUNK, chunk, 0)


@functools.cache
def _encode():
    return pl.kernel(_encode_body, **_kernel_kwargs())


def kernel(fields, sides, species, moves, items, abilities,
           move_attributes, pokemon_attributes,
           species_table, move_table, item_table, ability_table):
    sp = species.reshape(S).astype(jnp.int32)
    it = items.reshape(S).astype(jnp.int32)
    ab = abilities.reshape(S).astype(jnp.int32)
    mv = moves.reshape(S, 4).astype(jnp.int32)
    ma = move_attributes.reshape(S, 32)
    pa = pokemon_attributes.reshape(S, 32)
    spt = jnp.pad(species_table, ((0, V_SP - species_table.shape[0]), (0, 0)))
    mvt = jnp.pad(move_table, ((0, V_MV - move_table.shape[0]), (0, 0)))
    itt = jnp.pad(item_table, ((0, V_IT - item_table.shape[0]), (0, 0)))
    abt = jnp.pad(ability_table, ((0, V_AB - ability_table.shape[0]), (0, 0)))
    concat = _encode()(sp, it, ab, mv[:, 0], mv[:, 1], mv[:, 2], mv[:, 3],
                       ma, pa, spt, mvt, itt, abt)
    return (fields, sides, concat.reshape(B, 2, 6, D_OUT))
